# baseline (speedup 1.0000x reference)
import functools

import jax
import jax.numpy as jnp
from jax import lax
from jax.experimental import pallas as pl
from jax.experimental.pallas import tpu as pltpu
from jax.experimental.pallas import tpu_sc as plsc

NUM_USERS = 50000
NUM_ITEMS = 50000
NUM_NODES = NUM_USERS + NUM_ITEMS
DIM = 16
NUM_EDGES = 3200000
BATCH = 16384
NUM_LAYERS = 3

NC = 2
NS = 16
NW = NC * NS
SUB = 128
NSTR = 8
CHUNK = NSTR * SUB
STEPS = 100
EDGES_PAD = NW * STEPS * CHUNK
EROWS = EDGES_PAD // SUB
WROWS = STEPS * NSTR

ZB = 128
NBLK = (NUM_NODES + ZB - 1) // ZB
LAST = NUM_NODES - (NBLK - 1) * ZB

_mesh = plsc.VectorSubcoreMesh(core_axis_name="c", subcore_axis_name="s")
_params = pltpu.CompilerParams(use_tc_tiling_on_sc=False,
                               needs_layout_passes=False)


@functools.partial(
    pl.kernel,
    out_type=jax.ShapeDtypeStruct((NC * NUM_NODES, DIM), jnp.float32),
    mesh=_mesh,
    compiler_params=_params,
    scratch_types=[
        pltpu.VMEM_SHARED((NUM_NODES, DIM), jnp.float32),
        pltpu.VMEM((NSTR, SUB), jnp.int32),
        pltpu.VMEM((NSTR, SUB), jnp.int32),
        pltpu.VMEM((CHUNK,), jnp.float32),
        pltpu.VMEM((CHUNK, DIM), jnp.float32),
        pltpu.SemaphoreType.DMA,
    ],
)
def _layer(x_hbm, src_hbm, dst_hbm, w_hbm, out_hbm,
           y_sh, src_v, dst_v, w_v, rows_v, sem):
    c = lax.axis_index("c")
    s = lax.axis_index("s")
    wid = s * NC + c

    def _z(i, _):
        rows_v[i] = jnp.zeros((DIM,), jnp.float32)
        return 0
    lax.fori_loop(0, ZB, _z, 0)

    def _zcp(k, _):
        blk = k * NS + s

        @pl.when(blk < NBLK - 1)
        def _():
            pltpu.sync_copy(rows_v.at[pl.ds(0, ZB)],
                            y_sh.at[pl.ds(blk * ZB, ZB)])

        @pl.when(blk == NBLK - 1)
        def _():
            pltpu.sync_copy(rows_v.at[pl.ds(0, LAST)],
                            y_sh.at[pl.ds(blk * ZB, LAST)])
        return 0
    lax.fori_loop(0, (NBLK + NS - 1) // NS, _zcp, 0)
    plsc.subcore_barrier()

    erow0 = wid * WROWS

    def _step(i, _):
        row0 = erow0 + i * NSTR
        pltpu.sync_copy(src_hbm.at[pl.ds(row0, NSTR)], src_v)
        pltpu.sync_copy(dst_hbm.at[pl.ds(row0, NSTR)], dst_v)
        pltpu.sync_copy(w_hbm.at[pl.ds(row0 * SUB, CHUNK)], w_v)

        gats = [
            pltpu.async_copy(x_hbm.at[src_v.at[b]],
                             rows_v.at[pl.ds(b * SUB, SUB)], sem)
            for b in range(NSTR)
        ]
        for g in gats:
            g.wait()

        def _scale(g, _):
            w16 = w_v[pl.ds(g * 16, 16)]
            for k in range(16):
                j = g * 16 + k
                rows_v[j] = rows_v[j] * jnp.broadcast_to(w16[k], (DIM,))
            return 0
        lax.fori_loop(0, CHUNK // 16, _scale, 0)

        scats = [
            pltpu.async_copy(rows_v.at[pl.ds(b * SUB, SUB)],
                             y_sh.at[dst_v.at[b]], sem, add=True)
            for b in range(NSTR)
        ]
        for g in scats:
            g.wait()
        return 0

    lax.fori_loop(0, STEPS, _step, 0)
    plsc.subcore_barrier()

    def _wb(k, _):
        blk = k * NS + s

        @pl.when(blk < NBLK - 1)
        def _():
            r0 = blk * ZB
            pltpu.sync_copy(y_sh.at[pl.ds(r0, ZB)], rows_v.at[pl.ds(0, ZB)])
            pltpu.sync_copy(rows_v.at[pl.ds(0, ZB)],
                            out_hbm.at[pl.ds(c * NUM_NODES + r0, ZB)])

        @pl.when(blk == NBLK - 1)
        def _():
            r0 = blk * ZB
            pltpu.sync_copy(y_sh.at[pl.ds(r0, LAST)],
                            rows_v.at[pl.ds(0, LAST)])
            pltpu.sync_copy(rows_v.at[pl.ds(0, LAST)],
                            out_hbm.at[pl.ds(c * NUM_NODES + r0, LAST)])
        return 0
    lax.fori_loop(0, (NBLK + NS - 1) // NS, _wb, 0)


@functools.partial(
    pl.kernel,
    out_type=jax.ShapeDtypeStruct((NUM_NODES, DIM), jnp.float32),
    mesh=_mesh,
    compiler_params=_params,
    scratch_types=[
        pltpu.VMEM((ZB, DIM), jnp.float32),
        pltpu.VMEM((ZB, DIM), jnp.float32),
        pltpu.SemaphoreType.DMA,
    ],
)
def _merge(part_hbm, out_hbm, a_v, b_v, sem):
    c = lax.axis_index("c")
    s = lax.axis_index("s")
    wid = s * NC + c

    def _mg(k, _):
        blk = k * NW + wid

        @pl.when(blk < NBLK)
        def _():
            n = jnp.where(blk == NBLK - 1, LAST, ZB)
            r0 = blk * ZB
            ca = pltpu.async_copy(part_hbm.at[pl.ds(r0, n)], a_v.at[pl.ds(0, n)], sem)
            cb = pltpu.async_copy(part_hbm.at[pl.ds(NUM_NODES + r0, n)],
                                  b_v.at[pl.ds(0, n)], sem)
            ca.wait()
            cb.wait()

            def _add(j, _):
                a_v[j] = a_v[j] + b_v[j]
                return 0
            lax.fori_loop(0, n, _add, 0)
            pltpu.sync_copy(a_v.at[pl.ds(0, n)], out_hbm.at[pl.ds(r0, n)])
        return 0
    lax.fori_loop(0, (NBLK + NW - 1) // NW, _mg, 0)


PW = BATCH // NW
PROWS = PW // SUB


@functools.partial(
    pl.kernel,
    out_type=jax.ShapeDtypeStruct((BATCH,), jnp.float32),
    mesh=_mesh,
    compiler_params=_params,
    scratch_types=[
        pltpu.VMEM((PROWS, SUB), jnp.int32),
        pltpu.VMEM((PROWS, SUB), jnp.int32),
        pltpu.VMEM((PW, DIM), jnp.float32),
        pltpu.VMEM((PW, DIM), jnp.float32),
        pltpu.VMEM((PW, DIM), jnp.float32),
        pltpu.VMEM((PW, DIM), jnp.float32),
        pltpu.VMEM((PW,), jnp.float32),
        pltpu.SemaphoreType.DMA,
    ],
)
def _score(part_hbm, ui_hbm, ii_hbm, out_hbm,
           ui_v, ii_v, u0_v, u1_v, i0_v, i1_v, sc_v, sem):
    c = lax.axis_index("c")
    s = lax.axis_index("s")
    wid = s * NC + c

    for r in range(PROWS):
        pltpu.sync_copy(ui_hbm.at[pl.ds(wid * PW + r * SUB, SUB)], ui_v.at[r])
        pltpu.sync_copy(ii_hbm.at[pl.ds(wid * PW + r * SUB, SUB)], ii_v.at[r])

    def _off(r, _):
        for cc in range(8):
            sl = pl.ds(cc * 16, 16)
            ii_v[r, sl] = ii_v[r, sl] + NUM_USERS
        return 0
    lax.fori_loop(0, PROWS, _off, 0)

    cps = []
    for r in range(PROWS):
        cps.append(pltpu.async_copy(part_hbm.at[ui_v.at[r]],
                                    u0_v.at[pl.ds(r * SUB, SUB)], sem))
        cps.append(pltpu.async_copy(part_hbm.at[ii_v.at[r]],
                                    i0_v.at[pl.ds(r * SUB, SUB)], sem))
    for g in cps:
        g.wait()

    def _off2(r, _):
        for cc in range(8):
            sl = pl.ds(cc * 16, 16)
            ui_v[r, sl] = ui_v[r, sl] + NUM_NODES
            ii_v[r, sl] = ii_v[r, sl] + NUM_NODES
        return 0
    lax.fori_loop(0, PROWS, _off2, 0)

    cps = []
    for r in range(PROWS):
        cps.append(pltpu.async_copy(part_hbm.at[ui_v.at[r]],
                                    u1_v.at[pl.ds(r * SUB, SUB)], sem))
        cps.append(pltpu.async_copy(part_hbm.at[ii_v.at[r]],
                                    i1_v.at[pl.ds(r * SUB, SUB)], sem))
    for g in cps:
        g.wait()

    def _dot(g, _):
        pid = g * 16 + lax.iota(jnp.int32, 16)
        acc = jnp.zeros((16,), jnp.float32)
        for d in range(DIM):
            dd = jnp.full((16,), d, jnp.int32)
            u = (plsc.load_gather(u0_v, [pid, dd])
                 + plsc.load_gather(u1_v, [pid, dd]))
            it = (plsc.load_gather(i0_v, [pid, dd])
                  + plsc.load_gather(i1_v, [pid, dd]))
            acc = acc + u * it
        sc_v[pl.ds(g * 16, 16)] = acc
        return 0
    lax.fori_loop(0, PW // 16, _dot, 0)

    pltpu.sync_copy(sc_v, out_hbm.at[pl.ds(wid * PW, PW)])


def kernel(user_indices, item_indices, user_emb, item_emb, edge_index, edge_weight):
    x = jnp.concatenate([user_emb, item_emb], axis=0)

    pad = EDGES_PAD - NUM_EDGES
    src = jnp.concatenate([edge_index[0].astype(jnp.int32),
                           jnp.zeros((pad,), jnp.int32)])
    dst = jnp.concatenate([edge_index[1].astype(jnp.int32),
                           jnp.zeros((pad,), jnp.int32)])
    w = jnp.concatenate([edge_weight,
                         jnp.zeros((pad,), jnp.float32)])
    src2d = src.reshape(EROWS, SUB)
    dst2d = dst.reshape(EROWS, SUB)

    for layer in range(NUM_LAYERS):
        part = _layer(x, src2d, dst2d, w)
        if layer < NUM_LAYERS - 1:
            x = _merge(part)

    return _score(part, user_indices.astype(jnp.int32),
                  item_indices.astype(jnp.int32))
